# baseline (speedup 1.0000x reference)
import functools

import jax
import jax.numpy as jnp
from jax import lax
from jax.experimental import pallas as pl
from jax.experimental.pallas import tpu as pltpu
from jax.experimental.pallas import tpu_sc as plsc

_LINE_W = 4224
_SC_CHUNK = 256
_SC_W = 32 * _SC_CHUNK
_G_W = 4096

_GATHER_DN = lax.GatherDimensionNumbers(
    offset_dims=(), collapsed_slice_dims=(0,), start_index_map=(0,))


@functools.partial(
    pl.kernel,
    mesh=plsc.VectorSubcoreMesh(core_axis_name="c", subcore_axis_name="s"),
    out_type=jax.ShapeDtypeStruct((12, _SC_W), jnp.float32),
    scratch_types=[
        pltpu.VMEM((384,), jnp.float32),
        pltpu.VMEM((12, _SC_CHUNK), jnp.float32),
    ],
)
def _sc_line(wt_hbm, out_hbm, ws, linev):
    wid = lax.axis_index("s") * 2 + lax.axis_index("c")
    base = wid * _SC_CHUNK
    pltpu.sync_copy(wt_hbm, ws)
    for j in range(_SC_CHUNK // 16):
        t = lax.iota(jnp.int32, 16) + (base + j * 16)
        d = t - 2047
        n = jnp.abs(d)

        def stairs(off):
            v = jnp.full((16,), off + 8, jnp.int32)
            for th, lv in ((10, 9), (12, 10), (14, 11), (16, 12),
                           (20, 13), (23, 14), (27, 15)):
                v = jnp.where(n >= th, jnp.int32(off + lv), v)
            return v

        bpos = jnp.where(n < 8, n + 16, stairs(16))
        bneg = jnp.where(n < 8, n, stairs(0))
        bucket = jnp.where(d > 0, bpos, bneg)
        lo = bucket & 15
        is_hi = bucket >= 16
        for h in range(12):
            w0 = ws[pl.ds(h * 32, 16)]
            w1 = ws[pl.ds(h * 32 + 16, 16)]
            v0 = lax.gather(w0, lo[:, None], _GATHER_DN, (1,),
                            mode=lax.GatherScatterMode.PROMISE_IN_BOUNDS)
            v1 = lax.gather(w1, lo[:, None], _GATHER_DN, (1,),
                            mode=lax.GatherScatterMode.PROMISE_IN_BOUNDS)
            linev[h, pl.ds(j * 16, 16)] = jnp.where(is_hi, v1, v0)
    pltpu.sync_copy(linev, out_hbm.at[:, pl.ds(base, _SC_CHUNK)])


def _out_copy(lineg_ref, out_ref, sem_o, h, g):
    s = 2048 - 128 * (g + 1)
    return pltpu.make_async_copy(
        lineg_ref.at[h, :, pl.ds(s, 2048)],
        out_ref.at[0, h, pl.ds(128 * g, 128), :], sem_o)


def _expand_body(line_ref, out_ref, lineg_ref, sem_o):
    line = line_ref[:, :_LINE_W]
    for h in range(12):
        bcast = jnp.broadcast_to(line[h:h + 1, :], (128, _LINE_W))
        lineg_ref[h] = pltpu.roll(
            bcast, _LINE_W - 127, 1, stride=1, stride_axis=0)[:, :_G_W]
        for g in range(16):
            _out_copy(lineg_ref, out_ref, sem_o, h, g).start()
    for h in range(12):
        for g in range(16):
            _out_copy(lineg_ref, out_ref, sem_o, h, g).wait()


def kernel(qlen, klen, bc, W):
    del qlen, klen, bc
    line = _sc_line(W.T.reshape(-1))
    return pl.pallas_call(
        _expand_body,
        in_specs=[pl.BlockSpec(memory_space=pltpu.VMEM)],
        out_specs=pl.BlockSpec(memory_space=pl.ANY),
        out_shape=jax.ShapeDtypeStruct((1, 12, 2048, 2048), jnp.float32),
        scratch_shapes=[
            pltpu.VMEM((12, 128, _G_W), jnp.float32),
            pltpu.SemaphoreType.DMA,
        ],
    )(line)


# speedup vs baseline: 1.0078x; 1.0078x over previous
import functools

import jax
import jax.numpy as jnp
from jax import lax
from jax.experimental import pallas as pl
from jax.experimental.pallas import tpu as pltpu
from jax.experimental.pallas import tpu_sc as plsc

_LINE_W = 4224
_SC_CHUNK = 256
_SC_ACTIVE = 17
_SC_W = _SC_ACTIVE * _SC_CHUNK
_G_W = 4096

_GATHER_DN = lax.GatherDimensionNumbers(
    offset_dims=(), collapsed_slice_dims=(0,), start_index_map=(0,))


@functools.partial(
    pl.kernel,
    mesh=plsc.VectorSubcoreMesh(core_axis_name="c", subcore_axis_name="s"),
    out_type=jax.ShapeDtypeStruct((12, _SC_W), jnp.float32),
    scratch_types=[
        pltpu.VMEM((384,), jnp.float32),
        pltpu.VMEM((12, _SC_CHUNK), jnp.float32),
    ],
)
def _sc_line(wt_hbm, out_hbm, ws, linev):
    wid = lax.axis_index("s") * 2 + lax.axis_index("c")
    base = wid * _SC_CHUNK

    @pl.when(wid < _SC_ACTIVE)
    def _active():
        _sc_line_chunk(wt_hbm, out_hbm, ws, linev, base)


def _sc_line_chunk(wt_hbm, out_hbm, ws, linev, base):
    pltpu.sync_copy(wt_hbm, ws)
    for j in range(_SC_CHUNK // 16):
        t = lax.iota(jnp.int32, 16) + (base + j * 16)
        d = t - 2047
        n = jnp.abs(d)

        def stairs(off):
            v = jnp.full((16,), off + 8, jnp.int32)
            for th, lv in ((10, 9), (12, 10), (14, 11), (16, 12),
                           (20, 13), (23, 14), (27, 15)):
                v = jnp.where(n >= th, jnp.int32(off + lv), v)
            return v

        bpos = jnp.where(n < 8, n + 16, stairs(16))
        bneg = jnp.where(n < 8, n, stairs(0))
        bucket = jnp.where(d > 0, bpos, bneg)
        lo = bucket & 15
        is_hi = bucket >= 16
        for h in range(12):
            w0 = ws[pl.ds(h * 32, 16)]
            w1 = ws[pl.ds(h * 32 + 16, 16)]
            v0 = lax.gather(w0, lo[:, None], _GATHER_DN, (1,),
                            mode=lax.GatherScatterMode.PROMISE_IN_BOUNDS)
            v1 = lax.gather(w1, lo[:, None], _GATHER_DN, (1,),
                            mode=lax.GatherScatterMode.PROMISE_IN_BOUNDS)
            linev[h, pl.ds(j * 16, 16)] = jnp.where(is_hi, v1, v0)
    pltpu.sync_copy(linev, out_hbm.at[:, pl.ds(base, _SC_CHUNK)])


def _out_copy(lineg_ref, out_ref, sem_o, h, g):
    s = 2048 - 128 * (g + 1)
    return pltpu.make_async_copy(
        lineg_ref.at[h, :, pl.ds(s, 2048)],
        out_ref.at[0, h, pl.ds(128 * g, 128), :], sem_o)


def _expand_body(line_ref, out_ref, lineg_ref, sem_o):
    line = line_ref[:, :_LINE_W]
    for h in range(12):
        bcast = jnp.broadcast_to(line[h:h + 1, :], (128, _LINE_W))
        lineg_ref[h] = pltpu.roll(
            bcast, _LINE_W - 127, 1, stride=1, stride_axis=0)[:, :_G_W]
        for g in range(16):
            _out_copy(lineg_ref, out_ref, sem_o, h, g).start()
    for h in range(12):
        for g in range(16):
            _out_copy(lineg_ref, out_ref, sem_o, h, g).wait()


def kernel(qlen, klen, bc, W):
    del qlen, klen, bc
    line = _sc_line(W.T.reshape(-1))
    return pl.pallas_call(
        _expand_body,
        in_specs=[pl.BlockSpec(memory_space=pltpu.VMEM)],
        out_specs=pl.BlockSpec(memory_space=pl.ANY),
        out_shape=jax.ShapeDtypeStruct((1, 12, 2048, 2048), jnp.float32),
        scratch_shapes=[
            pltpu.VMEM((12, 128, _G_W), jnp.float32),
            pltpu.SemaphoreType.DMA,
        ],
    )(line)


# speedup vs baseline: 1.0340x; 1.0260x over previous
import functools

import jax
import jax.numpy as jnp
from jax import lax
from jax.experimental import pallas as pl
from jax.experimental.pallas import tpu as pltpu
from jax.experimental.pallas import tpu_sc as plsc

_LINE_W = 4224
_SC_CHUNK = 384
_SC_ACTIVE = 12
_SC_W = _SC_ACTIVE * _SC_CHUNK
_G_W = 4096

_GATHER_DN = lax.GatherDimensionNumbers(
    offset_dims=(), collapsed_slice_dims=(0,), start_index_map=(0,))


@functools.partial(
    pl.kernel,
    mesh=plsc.VectorSubcoreMesh(core_axis_name="c", subcore_axis_name="s",
                                num_cores=1),
    out_type=jax.ShapeDtypeStruct((12, _SC_W), jnp.float32),
    scratch_types=[
        pltpu.VMEM((384,), jnp.float32),
        pltpu.VMEM((12, _SC_CHUNK), jnp.float32),
    ],
)
def _sc_line(wt_hbm, out_hbm, ws, linev):
    wid = lax.axis_index("s")
    base = wid * _SC_CHUNK

    @pl.when(wid < _SC_ACTIVE)
    def _active():
        _sc_line_chunk(wt_hbm, out_hbm, ws, linev, base)


def _sc_line_chunk(wt_hbm, out_hbm, ws, linev, base):
    pltpu.sync_copy(wt_hbm, ws)
    for j in range(_SC_CHUNK // 16):
        t = lax.iota(jnp.int32, 16) + (base + j * 16)
        d = t - 2047
        n = jnp.abs(d)

        def stairs(off):
            v = jnp.full((16,), off + 8, jnp.int32)
            for th, lv in ((10, 9), (12, 10), (14, 11), (16, 12),
                           (20, 13), (23, 14), (27, 15)):
                v = jnp.where(n >= th, jnp.int32(off + lv), v)
            return v

        bpos = jnp.where(n < 8, n + 16, stairs(16))
        bneg = jnp.where(n < 8, n, stairs(0))
        bucket = jnp.where(d > 0, bpos, bneg)
        lo = bucket & 15
        is_hi = bucket >= 16
        for h in range(12):
            w0 = ws[pl.ds(h * 32, 16)]
            w1 = ws[pl.ds(h * 32 + 16, 16)]
            v0 = lax.gather(w0, lo[:, None], _GATHER_DN, (1,),
                            mode=lax.GatherScatterMode.PROMISE_IN_BOUNDS)
            v1 = lax.gather(w1, lo[:, None], _GATHER_DN, (1,),
                            mode=lax.GatherScatterMode.PROMISE_IN_BOUNDS)
            linev[h, pl.ds(j * 16, 16)] = jnp.where(is_hi, v1, v0)
    pltpu.sync_copy(linev, out_hbm.at[:, pl.ds(base, _SC_CHUNK)])


def _out_copy(lineg_ref, out_ref, sem_o, h, g):
    s = 2048 - 128 * (g + 1)
    return pltpu.make_async_copy(
        lineg_ref.at[h, :, pl.ds(s, 2048)],
        out_ref.at[0, h, pl.ds(128 * g, 128), :], sem_o)


def _expand_body(line_ref, out_ref, lineg_ref, sem_o):
    line = line_ref[:, :_LINE_W]
    for h in range(12):
        bcast = jnp.broadcast_to(line[h:h + 1, :], (128, _LINE_W))
        lineg_ref[h] = pltpu.roll(
            bcast, _LINE_W - 127, 1, stride=1, stride_axis=0)[:, :_G_W]
        for g in range(16):
            _out_copy(lineg_ref, out_ref, sem_o, h, g).start()
    for h in range(12):
        for g in range(16):
            _out_copy(lineg_ref, out_ref, sem_o, h, g).wait()


def kernel(qlen, klen, bc, W):
    del qlen, klen, bc
    line = _sc_line(W.T.reshape(-1))
    return pl.pallas_call(
        _expand_body,
        in_specs=[pl.BlockSpec(memory_space=pltpu.VMEM)],
        out_specs=pl.BlockSpec(memory_space=pl.ANY),
        out_shape=jax.ShapeDtypeStruct((1, 12, 2048, 2048), jnp.float32),
        scratch_shapes=[
            pltpu.VMEM((12, 128, _G_W), jnp.float32),
            pltpu.SemaphoreType.DMA,
        ],
    )(line)


# speedup vs baseline: 1.0954x; 1.0594x over previous
import functools

import jax
import jax.numpy as jnp
from jax import lax
from jax.experimental import pallas as pl
from jax.experimental.pallas import tpu as pltpu
from jax.experimental.pallas import tpu_sc as plsc

_LINE_W = 4224
_SC_CHUNK = 384
_SC_ACTIVE = 12
_SC_W = _SC_ACTIVE * _SC_CHUNK
_G_W = 4096

_GATHER_DN = lax.GatherDimensionNumbers(
    offset_dims=(), collapsed_slice_dims=(0,), start_index_map=(0,))

_OUT_SHAPE = jax.ShapeDtypeStruct((1, 12, 2048, 2048), jnp.float32)


def _bucket_rows(d):
    n = jnp.abs(d)
    base = jnp.where(d > 0, 16, 0).astype(jnp.int32)
    large = jnp.full_like(n, 8)
    for t in (10, 12, 14, 16, 20, 23, 27):
        large = large + (n >= t).astype(jnp.int32)
    return base + jnp.where(n < 8, n, large)


@functools.partial(
    pl.kernel,
    mesh=plsc.VectorSubcoreMesh(core_axis_name="c", subcore_axis_name="s",
                                num_cores=1),
    out_type=jax.ShapeDtypeStruct((12, _SC_W), jnp.float32),
    scratch_types=[
        pltpu.VMEM((384,), jnp.float32),
        pltpu.VMEM((12, _SC_CHUNK), jnp.float32),
    ],
)
def _sc_line(wt_hbm, out_hbm, ws, linev):
    wid = lax.axis_index("s")
    base = wid * _SC_CHUNK

    @pl.when(wid < _SC_ACTIVE)
    def _active():
        _sc_line_chunk(wt_hbm, out_hbm, ws, linev, base)


def _sc_line_chunk(wt_hbm, out_hbm, ws, linev, base):
    pltpu.sync_copy(wt_hbm, ws)
    for j in range(_SC_CHUNK // 16):
        t = lax.iota(jnp.int32, 16) + (base + j * 16)
        d = t - 2047
        n = jnp.abs(d)

        def stairs(off):
            v = jnp.full((16,), off + 8, jnp.int32)
            for th, lv in ((10, 9), (12, 10), (14, 11), (16, 12),
                           (20, 13), (23, 14), (27, 15)):
                v = jnp.where(n >= th, jnp.int32(off + lv), v)
            return v

        bpos = jnp.where(n < 8, n + 16, stairs(16))
        bneg = jnp.where(n < 8, n, stairs(0))
        bucket = jnp.where(d > 0, bpos, bneg)
        lo = bucket & 15
        is_hi = bucket >= 16
        for h in range(12):
            w0 = ws[pl.ds(h * 32, 16)]
            w1 = ws[pl.ds(h * 32 + 16, 16)]
            v0 = lax.gather(w0, lo[:, None], _GATHER_DN, (1,),
                            mode=lax.GatherScatterMode.PROMISE_IN_BOUNDS)
            v1 = lax.gather(w1, lo[:, None], _GATHER_DN, (1,),
                            mode=lax.GatherScatterMode.PROMISE_IN_BOUNDS)
            linev[h, pl.ds(j * 16, 16)] = jnp.where(is_hi, v1, v0)
    pltpu.sync_copy(linev, out_hbm.at[:, pl.ds(base, _SC_CHUNK)])


def _out_copy(lineg_ref, out_ref, sem_o, plane, h, g):
    s = 2048 - 128 * (g + 1)
    return pltpu.make_async_copy(
        lineg_ref.at[plane, :, pl.ds(s, 2048)],
        out_ref.at[0, h, pl.ds(128 * g, 128), :], sem_o)


def _expand_heads(line, heads, out_ref, lineg_ref, sem_o):
    for i, h in enumerate(heads):
        bcast = jnp.broadcast_to(line[h:h + 1, :], (128, _LINE_W))
        lineg_ref[i] = pltpu.roll(
            bcast, _LINE_W - 127, 1, stride=1, stride_axis=0)[:, :_G_W]
        for g in range(16):
            _out_copy(lineg_ref, out_ref, sem_o, i, h, g).start()
    for i, h in enumerate(heads):
        for g in range(16):
            _out_copy(lineg_ref, out_ref, sem_o, i, h, g).wait()


def _expand10_body(wt_ref, out_ref, lineg_ref, sem_o):
    t = jax.lax.broadcasted_iota(jnp.int32, (1, _LINE_W), 1)
    bucket = _bucket_rows(t - 2047)
    rows = jax.lax.broadcasted_iota(jnp.int32, (32, _LINE_W), 0)
    onehot = (rows == bucket).astype(jnp.float32)
    line = jax.lax.dot_general(
        wt_ref[...], onehot, (((1,), (0,)), ((), ())),
        preferred_element_type=jnp.float32,
        precision=jax.lax.Precision.HIGHEST)
    _expand_heads(line, tuple(range(10)), out_ref, lineg_ref, sem_o)


def _expand2_body(line_ref, part_ref, out_ref, lineg_ref, sem_o):
    del part_ref
    _expand_heads(line_ref[:, :_LINE_W], (10, 11), out_ref, lineg_ref, sem_o)


def kernel(qlen, klen, bc, W):
    del qlen, klen, bc
    wt = W.T
    line = _sc_line(wt.reshape(-1))
    part = pl.pallas_call(
        _expand10_body,
        in_specs=[pl.BlockSpec(memory_space=pltpu.VMEM)],
        out_specs=pl.BlockSpec(memory_space=pl.ANY),
        out_shape=_OUT_SHAPE,
        scratch_shapes=[
            pltpu.VMEM((10, 128, _G_W), jnp.float32),
            pltpu.SemaphoreType.DMA,
        ],
    )(wt)
    return pl.pallas_call(
        _expand2_body,
        in_specs=[pl.BlockSpec(memory_space=pltpu.VMEM),
                  pl.BlockSpec(memory_space=pl.ANY)],
        out_specs=pl.BlockSpec(memory_space=pl.ANY),
        out_shape=_OUT_SHAPE,
        input_output_aliases={1: 0},
        scratch_shapes=[
            pltpu.VMEM((2, 128, _G_W), jnp.float32),
            pltpu.SemaphoreType.DMA,
        ],
    )(line, part)
